# Initial kernel scaffold; baseline (speedup 1.0000x reference)
import jax
import jax.numpy as jnp
from jax.experimental import pallas as pl


def kernel(inputs, V, targets, label_to_pairs, indexs):
    raise NotImplementedError("write your pallas kernel here")


# speedup vs baseline: 1.2878x; 1.2878x over previous
import functools

import jax
import jax.numpy as jnp
from jax.experimental import pallas as pl
from jax.experimental.pallas import tpu as pltpu

_MARGIN = 0.3
_TILE_C = 2048


def _softplus(x):
    return jnp.logaddexp(x, 0.0)


def _exloss_kernel(x_ref, v_ref, tgt_ref, pairs_ref,
                   out_ref, loss_ref,
                   m_ref, s_ref, tl_ref, th_ref, invn_ref,
                   *, C, P, margin):
    j = pl.program_id(0)
    nj = pl.num_programs(0)
    B = x_ref.shape[0]
    Ct = v_ref.shape[0]

    x = x_ref[...]

    @pl.when(j == 0)
    def _init():
        m_ref[...] = jnp.full(m_ref.shape, -1e30, jnp.float32)
        s_ref[...] = jnp.zeros(s_ref.shape, jnp.float32)
        tl_ref[...] = jnp.zeros(tl_ref.shape, jnp.float32)
        th_ref[...] = jnp.zeros(th_ref.shape, jnp.float32)
        invn_ref[...] = jax.lax.rsqrt(
            jnp.maximum(jnp.sum(x * x, axis=1, keepdims=True), 1e-24))

    block = jax.lax.dot_general(x, v_ref[...], (((1,), (1,)), ((), ())),
                                preferred_element_type=jnp.float32)
    out_ref[...] = block

    cols = j * Ct + jax.lax.broadcasted_iota(jnp.int32, (1, Ct), 1)
    colmask = cols < C
    tgt = tgt_ref[...]
    tmask = cols == tgt

    blk = jnp.where(colmask, block, -1e30)
    m_old = m_ref[...]
    m_new = jnp.maximum(m_old, jnp.max(blk, axis=1, keepdims=True))
    p = jnp.exp(blk - m_new)
    s_ref[...] = s_ref[...] * jnp.exp(m_old - m_new) + jnp.sum(
        p, axis=1, keepdims=True)
    m_ref[...] = m_new
    tl_ref[...] += jnp.sum(jnp.where(tmask, block, 0.0), axis=1, keepdims=True)

    invn = invn_ref[...]
    tsims = block * invn
    pos_cnt = jnp.sum(tmask.astype(jnp.float32), axis=0, keepdims=True)
    has_pos = pos_cnt > 0.0
    thpsim_raw = jnp.min(jnp.where(tmask, tsims, 1e30), axis=0, keepdims=True)
    thpsim = jnp.where(has_pos, thpsim_raw, 0.0)
    thp = jnp.where(has_pos, _softplus(-thpsim), 0.0)
    tthrd = jnp.where(has_pos, thpsim - margin, 1.0 - margin)
    tsel = jnp.logical_and(jnp.logical_not(tmask), tsims > tthrd)
    tself = tsel.astype(jnp.float32)
    tcnt = jnp.sum(tself, axis=0, keepdims=True)
    tsum = jnp.sum(jnp.where(tsel, _softplus(tsims), 0.0), axis=0,
                   keepdims=True)
    thn = jnp.where(tcnt > 0.0, tsum / jnp.maximum(tcnt, 1.0), 0.0)
    th_ref[...] += jnp.where(colmask, thp + thn, 0.0)

    @pl.when(j == nj - 1)
    def _finish():
        bu = jnp.mean(m_ref[...] + jnp.log(s_ref[...]) - tl_ref[...])
        th_loss = jnp.sum(th_ref[...]) / C

        ninp = x * invn_ref[...]
        sim = jax.lax.dot_general(ninp, ninp, (((1,), (1,)), ((), ())),
                                  preferred_element_type=jnp.float32)
        colid = jax.lax.broadcasted_iota(jnp.int32, (1, B), 1)
        pairs = pairs_ref[...]
        hp = jnp.full((B, 1), 2.0, jnp.float32)
        for q in range(P):
            pid = pairs[:, q:q + 1]
            ps = jnp.sum(jnp.where(colid == pid, sim, 0.0), axis=1,
                         keepdims=True)
            hp = jnp.minimum(hp, ps)
        thr = hp - margin
        cnt = jnp.zeros((B, 1), jnp.float32)
        nsum = jnp.zeros((B, 1), jnp.float32)
        for q in range(P):
            nid = pairs[:, P + q:P + q + 1]
            ns = jnp.sum(jnp.where(colid == nid, sim, 0.0), axis=1,
                         keepdims=True)
            sel = ns > thr
            cnt += sel.astype(jnp.float32)
            nsum += jnp.where(sel, _softplus(ns), 0.0)
        hn = jnp.where(cnt > 0.0, nsum / jnp.maximum(cnt, 1.0), 0.0)
        h_loss = jnp.mean(_softplus(-hp) + hn)

        loss_ref[...] = jnp.full(loss_ref.shape, bu + h_loss + th_loss,
                                 jnp.float32)


def _run(inputs, V, tgt2, pairs, tile_c, interpret=False):
    B, D = inputs.shape
    C = V.shape[0]
    P = pairs.shape[1] // 2
    grid = pl.cdiv(C, tile_c)
    body = functools.partial(_exloss_kernel, C=C, P=P, margin=_MARGIN)
    out, loss = pl.pallas_call(
        body,
        grid=(grid,),
        in_specs=[
            pl.BlockSpec((B, D), lambda j: (0, 0)),
            pl.BlockSpec((tile_c, D), lambda j: (j, 0)),
            pl.BlockSpec((B, 1), lambda j: (0, 0)),
            pl.BlockSpec((B, 2 * P), lambda j: (0, 0)),
        ],
        out_specs=[
            pl.BlockSpec((B, tile_c), lambda j: (0, j)),
            pl.BlockSpec((8, 128), lambda j: (0, 0)),
        ],
        out_shape=[
            jax.ShapeDtypeStruct((B, C), jnp.float32),
            jax.ShapeDtypeStruct((8, 128), jnp.float32),
        ],
        scratch_shapes=[
            pltpu.VMEM((B, 1), jnp.float32),
            pltpu.VMEM((B, 1), jnp.float32),
            pltpu.VMEM((B, 1), jnp.float32),
            pltpu.VMEM((1, tile_c), jnp.float32),
            pltpu.VMEM((B, 1), jnp.float32),
        ],
        compiler_params=pltpu.CompilerParams(
            dimension_semantics=("arbitrary",)),
        interpret=interpret,
    )(inputs, V, tgt2, pairs)
    return out, loss


def kernel(inputs, V, targets, label_to_pairs, indexs):
    B, D = inputs.shape
    C = V.shape[0]
    P = label_to_pairs.shape[2]
    tgt2 = targets.astype(jnp.int32).reshape(B, 1)
    pairs = label_to_pairs.astype(jnp.int32).reshape(B, 2 * P)
    out, loss = _run(inputs, V, tgt2, pairs, _TILE_C)
    return loss[0, 0], out


# speedup vs baseline: 1.6678x; 1.2951x over previous
import functools

import jax
import jax.numpy as jnp
from jax.experimental import pallas as pl
from jax.experimental.pallas import tpu as pltpu

_MARGIN = 0.3
_TILE_C = 2048

_SP_C0 = 0.6931473570802212
_SP_C1 = 0.12499416966835278
_SP_C2 = -0.005178683812392345
_SP_C3 = 0.00029877731655706833


def _softplus_poly(x):
    u = x * x
    p = (_SP_C3 * u + _SP_C2) * u + _SP_C1
    return (p * u + _SP_C0) + 0.5 * x


def _softplus(x):
    return jnp.logaddexp(x, 0.0)


def _exloss_kernel(x_ref, v_ref, tgt_ref, pairs_ref,
                   out_ref, loss_ref,
                   s_ref, tl_ref, th_ref, invn_ref,
                   *, C, P, margin, npad):
    j = pl.program_id(0)
    nj = pl.num_programs(0)
    B = x_ref.shape[0]
    Ct = v_ref.shape[0]

    x = x_ref[...]

    @pl.when(j == 0)
    def _init():
        s_ref[...] = jnp.zeros(s_ref.shape, jnp.float32)
        tl_ref[...] = jnp.zeros(tl_ref.shape, jnp.float32)
        th_ref[...] = jnp.zeros(th_ref.shape, jnp.float32)
        invn_ref[...] = jax.lax.rsqrt(
            jnp.maximum(jnp.sum(x * x, axis=1, keepdims=True), 1e-24))

    block = jax.lax.dot_general(x, v_ref[...], (((1,), (1,)), ((), ())),
                                preferred_element_type=jnp.float32)
    out_ref[...] = block

    cols = j * Ct + jax.lax.broadcasted_iota(jnp.int32, (1, Ct), 1)
    tmask = cols == tgt_ref[...]
    tmask_f = tmask.astype(jnp.float32)

    s_ref[...] += jnp.sum(jnp.exp(block), axis=1, keepdims=True)
    tl_ref[...] += jnp.sum(tmask_f * block, axis=1, keepdims=True)

    invn = invn_ref[...]
    tsims = block * invn
    thpsim_raw = jnp.min(jnp.where(tmask, tsims, 1e30), axis=0, keepdims=True)
    has_pos = thpsim_raw < 1e29
    thpsim = jnp.where(has_pos, thpsim_raw, 0.0)
    tthrd = jnp.where(has_pos, thpsim - margin, 1.0 - margin)
    self_f = (tsims > tthrd).astype(jnp.float32) - tmask_f
    tcnt = jnp.sum(self_f, axis=0, keepdims=True)
    tsum = jnp.sum(self_f * _softplus_poly(tsims), axis=0, keepdims=True)
    thn = jnp.where(tcnt > 0.0, tsum / jnp.maximum(tcnt, 1.0), 0.0)
    thp = jnp.where(has_pos, _softplus(-thpsim), 0.0)
    th_ref[...] += thp + thn

    @pl.when(j == nj - 1)
    def _finish():
        bu = jnp.mean(jnp.log(s_ref[...] - float(npad)) - tl_ref[...])
        th_loss = jnp.sum(th_ref[...]) / C

        ninp = x * invn_ref[...]
        sim = jax.lax.dot_general(ninp, ninp, (((1,), (1,)), ((), ())),
                                  preferred_element_type=jnp.float32)
        colid = jax.lax.broadcasted_iota(jnp.int32, (1, B), 1)
        pairs = pairs_ref[...]
        hp = jnp.full((B, 1), 2.0, jnp.float32)
        for q in range(P):
            pid = pairs[:, q:q + 1]
            ps = jnp.sum(jnp.where(colid == pid, sim, 0.0), axis=1,
                         keepdims=True)
            hp = jnp.minimum(hp, ps)
        thr = hp - margin
        cnt = jnp.zeros((B, 1), jnp.float32)
        nsum = jnp.zeros((B, 1), jnp.float32)
        for q in range(P):
            nid = pairs[:, P + q:P + q + 1]
            ns = jnp.sum(jnp.where(colid == nid, sim, 0.0), axis=1,
                         keepdims=True)
            sel = ns > thr
            cnt += sel.astype(jnp.float32)
            nsum += jnp.where(sel, _softplus(ns), 0.0)
        hn = jnp.where(cnt > 0.0, nsum / jnp.maximum(cnt, 1.0), 0.0)
        h_loss = jnp.mean(_softplus(-hp) + hn)

        loss_ref[...] = jnp.full(loss_ref.shape, bu + h_loss + th_loss,
                                 jnp.float32)


def _run(inputs, V, tgt2, pairs, tile_c, interpret=False):
    B, D = inputs.shape
    C = V.shape[0]
    P = pairs.shape[1] // 2
    grid = pl.cdiv(C, tile_c)
    npad = grid * tile_c - C
    Vp = jnp.concatenate(
        [V, jnp.zeros((npad, D), jnp.float32)]) if npad else V
    body = functools.partial(_exloss_kernel, C=C, P=P, margin=_MARGIN,
                             npad=npad)
    out, loss = pl.pallas_call(
        body,
        grid=(grid,),
        in_specs=[
            pl.BlockSpec((B, D), lambda j: (0, 0)),
            pl.BlockSpec((tile_c, D), lambda j: (j, 0)),
            pl.BlockSpec((B, 1), lambda j: (0, 0)),
            pl.BlockSpec((B, 2 * P), lambda j: (0, 0)),
        ],
        out_specs=[
            pl.BlockSpec((B, tile_c), lambda j: (0, j)),
            pl.BlockSpec((8, 128), lambda j: (0, 0)),
        ],
        out_shape=[
            jax.ShapeDtypeStruct((B, C), jnp.float32),
            jax.ShapeDtypeStruct((8, 128), jnp.float32),
        ],
        scratch_shapes=[
            pltpu.VMEM((B, 1), jnp.float32),
            pltpu.VMEM((B, 1), jnp.float32),
            pltpu.VMEM((1, tile_c), jnp.float32),
            pltpu.VMEM((B, 1), jnp.float32),
        ],
        compiler_params=pltpu.CompilerParams(
            dimension_semantics=("arbitrary",)),
        interpret=interpret,
    )(inputs, Vp, tgt2, pairs)
    return out, loss


def kernel(inputs, V, targets, label_to_pairs, indexs):
    B, D = inputs.shape
    P = label_to_pairs.shape[2]
    tgt2 = targets.astype(jnp.int32).reshape(B, 1)
    pairs = label_to_pairs.astype(jnp.int32).reshape(B, 2 * P)
    out, loss = _run(inputs, V, tgt2, pairs, _TILE_C)
    return loss[0, 0], out
